# baseline (speedup 1.0000x reference)
import jax
import jax.numpy as jnp
from jax import lax
from jax.experimental import pallas as pl
from jax.experimental.pallas import tpu as pltpu
from jax.experimental.pallas import tpu_sc as plsc

NUM_EMB = 1_000_000
DIM = 64
OUT_DIM = 128
BATCH = 16384

NC, NS = 2, 16
NW = NC * NS
B_PER_W = BATCH // NW
CHUNK = 128
NCHUNK = B_PER_W // CHUNK


def _gather_kernel(idx_hbm, table_hbm, out_hbm, idx_v, rows_v, sem):
    wid = lax.axis_index("s") * NC + lax.axis_index("c")
    base = wid * B_PER_W
    pltpu.sync_copy(idx_hbm.at[pl.ds(wid * NCHUNK, NCHUNK)], idx_v)
    copies = []
    for c in range(NCHUNK):
        copies.append(
            pltpu.make_async_copy(
                table_hbm.at[idx_v.at[c]],
                rows_v.at[pl.ds(c * CHUNK, CHUNK)],
                sem,
            )
        )
    for cp in copies:
        cp.start()
    for cp in copies:
        cp.wait()
    pltpu.sync_copy(rows_v, out_hbm.at[pl.ds(base, B_PER_W)])


@jax.jit
def _sc_gather(idx2d, table):
    mesh = plsc.VectorSubcoreMesh(core_axis_name="c", subcore_axis_name="s")
    return pl.kernel(
        _gather_kernel,
        out_type=jax.ShapeDtypeStruct((BATCH, DIM), jnp.float32),
        mesh=mesh,
        scratch_types=[
            pltpu.VMEM((NCHUNK, CHUNK), jnp.int32),
            pltpu.VMEM((B_PER_W, DIM), jnp.float32),
            pltpu.SemaphoreType.DMA,
        ],
        compiler_params=pltpu.CompilerParams(use_tc_tiling_on_sc=False),
    )(idx2d, table)


BLK = 2048


def _mm_kernel(g_ref, wt_ref, b_ref, o_ref):
    h = jnp.maximum(g_ref[...], 0.0)
    o_ref[...] = (
        jnp.dot(h, wt_ref[...], preferred_element_type=jnp.float32) + b_ref[...]
    )


@jax.jit
def _tc_head(g, wt, b2d):
    return pl.pallas_call(
        _mm_kernel,
        grid=(BATCH // BLK,),
        in_specs=[
            pl.BlockSpec((BLK, DIM), lambda i: (i, 0)),
            pl.BlockSpec((DIM, OUT_DIM), lambda i: (0, 0)),
            pl.BlockSpec((1, OUT_DIM), lambda i: (0, 0)),
        ],
        out_specs=pl.BlockSpec((BLK, OUT_DIM), lambda i: (i, 0)),
        out_shape=jax.ShapeDtypeStruct((BATCH, OUT_DIM), jnp.float32),
    )(g, wt, b2d)


def kernel(env_index, table, W, b):
    idx2d = env_index.astype(jnp.int32).reshape(BATCH // CHUNK, CHUNK)
    g = _sc_gather(idx2d, table)
    return _tc_head(g, W.T, b.reshape(1, OUT_DIM))


# speedup vs baseline: 2.7136x; 2.7136x over previous
import jax
import jax.numpy as jnp
from jax import lax
from jax.experimental import pallas as pl
from jax.experimental.pallas import tpu as pltpu
from jax.experimental.pallas import tpu_sc as plsc

NUM_EMB = 1_000_000
DIM = 64
OUT_DIM = 128
BATCH = 16384
PADDED = 128

NC, NS = 2, 16
NW = NC * NS
B_PER_W = BATCH // NW
GRP = 2
NSLOT = 4
NGRP = B_PER_W // GRP


def _gather_kernel(idx_hbm, tableT_hbm, out_hbm, idx_v, idx_s, chunks, rows_v, sem):
    wid = lax.axis_index("s") * NC + lax.axis_index("c")
    base = wid * B_PER_W
    pltpu.sync_copy(idx_hbm.at[pl.ds(base, B_PER_W)], idx_v)

    def stage(s, carry):
        iv = idx_v[pl.ds(16 * s, 16)]
        for t in range(16):
            idx_s[16 * s + t] = iv[t]
        return carry

    lax.fori_loop(0, B_PER_W // 16, stage, 0)

    def blk_copy(k, slot):
        c = pl.multiple_of((idx_s[k] >> 7) << 7, 128)
        return pltpu.make_async_copy(
            tableT_hbm.at[:, pl.ds(c, 128)],
            chunks.at[slot],
            sem,
        )

    def fire(g):
        half = (g % 2) * GRP
        for b in range(GRP):
            blk_copy(g * GRP + b, half + b).start()

    def extract(g):
        half = (g % 2) * GRP
        for b in range(GRP):
            k = g * GRP + b
            blk_copy(k, half + b).wait()
            l = idx_s[k] & 127
            lvec = jnp.full((16,), l, jnp.int32)
            for m in range(DIM // 16):
                jvec = lax.iota(jnp.int32, 16) + (16 * m)
                vals = plsc.load_gather(chunks.at[half + b], [jvec, lvec])
                rows_v[k, pl.ds(16 * m, 16)] = vals

    def body(g, carry):
        fire(g)
        extract(g - 1)
        return carry

    fire(0)
    lax.fori_loop(1, NGRP, body, 0)
    extract(NGRP - 1)
    pltpu.sync_copy(rows_v, out_hbm.at[pl.ds(base, B_PER_W)])


@jax.jit
def _sc_gather(idx, tableT):
    mesh = plsc.VectorSubcoreMesh(core_axis_name="c", subcore_axis_name="s")
    return pl.kernel(
        _gather_kernel,
        out_type=jax.ShapeDtypeStruct((BATCH, PADDED), jnp.float32),
        mesh=mesh,
        scratch_types=[
            pltpu.VMEM((B_PER_W,), jnp.int32),
            pltpu.SMEM((B_PER_W,), jnp.int32),
            pltpu.VMEM((NSLOT, DIM, 128), jnp.float32),
            pltpu.VMEM((B_PER_W, PADDED), jnp.float32),
            pltpu.SemaphoreType.DMA,
        ],
        compiler_params=pltpu.CompilerParams(
            use_tc_tiling_on_sc=True, needs_layout_passes=False
        ),
    )(idx, tableT)


BLK = 2048


def _mm_kernel(g_ref, wt_ref, b_ref, o_ref):
    h = jnp.maximum(g_ref[:, :DIM], 0.0)
    o_ref[...] = (
        jnp.dot(h, wt_ref[...], preferred_element_type=jnp.float32) + b_ref[...]
    )


@jax.jit
def _tc_head(g, wt, b2d):
    return pl.pallas_call(
        _mm_kernel,
        grid=(BATCH // BLK,),
        in_specs=[
            pl.BlockSpec((BLK, PADDED), lambda i: (i, 0)),
            pl.BlockSpec((DIM, OUT_DIM), lambda i: (0, 0)),
            pl.BlockSpec((1, OUT_DIM), lambda i: (0, 0)),
        ],
        out_specs=pl.BlockSpec((BLK, OUT_DIM), lambda i: (i, 0)),
        out_shape=jax.ShapeDtypeStruct((BATCH, OUT_DIM), jnp.float32),
    )(g, wt, b2d)


def kernel(env_index, table, W, b):
    idx = env_index.astype(jnp.int32)
    g = _sc_gather(idx, table.T)
    return _tc_head(g, W.T, b.reshape(1, OUT_DIM))


# speedup vs baseline: 3.0748x; 1.1331x over previous
import jax
import jax.numpy as jnp
from jax import lax
from jax.experimental import pallas as pl
from jax.experimental.pallas import tpu as pltpu
from jax.experimental.pallas import tpu_sc as plsc

NUM_EMB = 1_000_000
DIM = 64
OUT_DIM = 128
BATCH = 16384
PADDED = 128

NC, NS = 2, 16
NW = NC * NS
B_PER_W = BATCH // NW
NSLOT = 5


def _gather_kernel(idx_hbm, tableT_hbm, out_hbm, idx_v, idx_s, chunks, rows_v, sem):
    wid = lax.axis_index("s") * NC + lax.axis_index("c")
    base = wid * B_PER_W
    pltpu.sync_copy(idx_hbm.at[pl.ds(base, B_PER_W)], idx_v)

    def stage(s, carry):
        iv = idx_v[pl.ds(16 * s, 16)]
        for t in range(16):
            idx_s[16 * s + t] = iv[t]
        return carry

    lax.fori_loop(0, B_PER_W // 16, stage, 0)

    def blk_copy(k):
        c = pl.multiple_of((idx_s[k] >> 7) << 7, 128)
        return pltpu.make_async_copy(
            tableT_hbm.at[:, pl.ds(c, 128)],
            chunks.at[k % NSLOT],
            sem,
        )

    def extract(k):
        blk_copy(k).wait()
        l = idx_s[k] & 127
        lvec = jnp.full((16,), l, jnp.int32)
        for m in range(DIM // 16):
            jvec = lax.iota(jnp.int32, 16) + (16 * m)
            vals = plsc.load_gather(chunks.at[k % NSLOT], [jvec, lvec])
            rows_v[k, pl.ds(16 * m, 16)] = vals

    def prime(k, carry):
        blk_copy(k).start()
        return carry

    def body(k, carry):
        blk_copy(k).start()
        extract(k - NSLOT)
        return carry

    def tail(k, carry):
        extract(k)
        return carry

    lax.fori_loop(0, NSLOT, prime, 0)
    lax.fori_loop(NSLOT, B_PER_W, body, 0)
    lax.fori_loop(B_PER_W - NSLOT, B_PER_W, tail, 0)
    pltpu.sync_copy(rows_v, out_hbm.at[pl.ds(base, B_PER_W)])


@jax.jit
def _sc_gather(idx, tableT):
    mesh = plsc.VectorSubcoreMesh(core_axis_name="c", subcore_axis_name="s")
    return pl.kernel(
        _gather_kernel,
        out_type=jax.ShapeDtypeStruct((BATCH, PADDED), jnp.float32),
        mesh=mesh,
        scratch_types=[
            pltpu.VMEM((B_PER_W,), jnp.int32),
            pltpu.SMEM((B_PER_W,), jnp.int32),
            pltpu.VMEM((NSLOT, DIM, 128), jnp.float32),
            pltpu.VMEM((B_PER_W, PADDED), jnp.float32),
            pltpu.SemaphoreType.DMA,
        ],
        compiler_params=pltpu.CompilerParams(
            use_tc_tiling_on_sc=True, needs_layout_passes=False
        ),
    )(idx, tableT)


BLK = 2048


def _mm_kernel(g_ref, wt_ref, b_ref, o_ref):
    h = jnp.maximum(g_ref[:, :DIM], 0.0)
    o_ref[...] = (
        jnp.dot(h, wt_ref[...], preferred_element_type=jnp.float32) + b_ref[...]
    )


@jax.jit
def _tc_head(g, wt, b2d):
    return pl.pallas_call(
        _mm_kernel,
        grid=(BATCH // BLK,),
        in_specs=[
            pl.BlockSpec((BLK, PADDED), lambda i: (i, 0)),
            pl.BlockSpec((DIM, OUT_DIM), lambda i: (0, 0)),
            pl.BlockSpec((1, OUT_DIM), lambda i: (0, 0)),
        ],
        out_specs=pl.BlockSpec((BLK, OUT_DIM), lambda i: (i, 0)),
        out_shape=jax.ShapeDtypeStruct((BATCH, OUT_DIM), jnp.float32),
    )(g, wt, b2d)


def kernel(env_index, table, W, b):
    idx = env_index.astype(jnp.int32)
    g = _sc_gather(idx, table.T)
    return _tc_head(g, W.T, b.reshape(1, OUT_DIM))
